# baseline (speedup 1.0000x reference)
import functools

import jax
import jax.numpy as jnp
from jax import lax
from jax.experimental import pallas as pl
from jax.experimental.pallas import tpu as pltpu
from jax.experimental.pallas import tpu_sc as plsc

N = 10000
E = 320000
D = 128
B = 8
NPG = N // B
NC = 2
NS = 16
NW = NC * NS
EPW = E // NW
EPWP = 10240
K = 64
NCHUNK = EPWP // K
NBUF = 4
NGRP = NCHUNK // NBUF
NB2 = NGRP // 2
NPAD = 10240
RPT = NPAD // NS
RB = 128
R = 1000
G = N // R


def _sc_mesh():
    return plsc.VectorSubcoreMesh(core_axis_name="c", subcore_axis_name="s")


def _sc_degree(cmb4, onecol, z_rows):
    @functools.partial(
        pl.kernel,
        out_type=jax.ShapeDtypeStruct((NC, NPAD, D), jnp.float32),
        mesh=_sc_mesh(),
        scratch_types=[
            pltpu.VMEM((K, D), jnp.float32),
            pltpu.VMEM((2 * NBUF, K), jnp.int32),
            pltpu.VMEM((2 * NBUF, K), jnp.int32),
            pltpu.VMEM_SHARED((NPAD, D), jnp.float32),
            pltpu.SemaphoreType.DMA,
        ]
        + [pltpu.SemaphoreType.DMA for _ in range(NBUF)],
    )
    def deg_kernel(cmb_h, oc_h, z_h, out_h, cbuf, ib0, ib1, acc, isem, *sems):
        c = lax.axis_index("c")
        s = lax.axis_index("s")
        wid = c * NS + s
        pltpu.sync_copy(oc_h, cbuf)
        base = s * RPT
        for j in range(RPT // RB):
            pltpu.sync_copy(z_h, acc.at[pl.ds(base + j * RB, RB)])
        plsc.subcore_barrier()

        def body(t, carry):
            pltpu.sync_copy(cmb_h.at[wid, 2 * t], ib0)
            dib1 = pltpu.async_copy(cmb_h.at[wid, 2 * t + 1], ib1, isem)
            d1 = [pltpu.async_copy(cbuf, acc.at[ib0.at[NBUF + b]], sems[b],
                                   add=True) for b in range(NBUF)]
            dib1.wait()
            for b in range(NBUF):
                d1[b].wait()
            d2 = [pltpu.async_copy(cbuf, acc.at[ib1.at[NBUF + b]], sems[b],
                                   add=True) for b in range(NBUF)]
            for b in range(NBUF):
                d2[b].wait()
            return carry

        lax.fori_loop(0, NB2, body, 0)
        plsc.subcore_barrier()
        pltpu.sync_copy(acc.at[pl.ds(base, RPT)], out_h.at[c, pl.ds(base, RPT)])

    return deg_kernel(cmb4, onecol, z_rows)


def _sc_scatter(g, cmb4, z_rows):
    @functools.partial(
        pl.kernel,
        out_type=jax.ShapeDtypeStruct((NC, NPAD, D), jnp.float32),
        mesh=_sc_mesh(),
        scratch_types=[
            pltpu.VMEM((2 * NBUF, K), jnp.int32),
            pltpu.VMEM((2 * NBUF, K), jnp.int32),
            pltpu.VMEM_SHARED((NPAD, D), jnp.float32),
            pltpu.SemaphoreType.DMA,
        ]
        + [pltpu.VMEM((K, D), jnp.float32) for _ in range(NBUF)]
        + [pltpu.SemaphoreType.DMA for _ in range(NBUF)],
    )
    def scat_kernel(g_h, cmb_h, zr_h, out_h, ib0, ib1, acc, isem, *scr):
        rows = scr[:NBUF]
        sems = scr[NBUF:]
        c = lax.axis_index("c")
        s = lax.axis_index("s")
        wid = c * NS + s
        base = s * RPT
        for j in range(RPT // RB):
            pltpu.sync_copy(zr_h, acc.at[pl.ds(base + j * RB, RB)])
        plsc.subcore_barrier()

        def body(t, carry):
            for half, ib in ((0, ib0), (1, ib1)):
                pltpu.sync_copy(cmb_h.at[wid, 2 * t + half], ib)
                ga = [pltpu.async_copy(g_h.at[ib.at[b]], rows[b], sems[b])
                      for b in range(NBUF)]
                for b in range(NBUF):
                    ga[b].wait()
                sa = [pltpu.async_copy(rows[b], acc.at[ib.at[NBUF + b]],
                                       sems[b], add=True)
                      for b in range(NBUF)]
                for b in range(NBUF):
                    sa[b].wait()
            return carry

        lax.fori_loop(0, NB2, body, 0)
        plsc.subcore_barrier()
        pltpu.sync_copy(acc.at[pl.ds(base, RPT)], out_h.at[c, pl.ds(base, RPT)])

    return scat_kernel(g, cmb4, z_rows)


def _tc_pre(x, W_in, b_in, counts_t, Wg1):
    def body(x_r, wi_r, bi_r, ct_r, wg_r, g_r, db_r):
        h = jnp.maximum(x_r[...] @ wi_r[...] + bi_r[...][None, :], 0.0)
        ct = ct_r[...]
        dcol = lax.rsqrt(ct[:, 0:1] + ct[:, 1:2] + 1.0)
        db_r[...] = jnp.broadcast_to(dcol, (R, D))
        g_r[...] = (h @ wg_r[...]) * dcol

    return pl.pallas_call(
        body,
        grid=(G,),
        in_specs=[
            pl.BlockSpec((R, D), lambda i: (i, 0)),
            pl.BlockSpec((D, D), lambda i: (0, 0)),
            pl.BlockSpec((D,), lambda i: (0,)),
            pl.BlockSpec((R, NC), lambda i: (i, 0)),
            pl.BlockSpec((D, D), lambda i: (0, 0)),
        ],
        out_specs=[
            pl.BlockSpec((R, D), lambda i: (i, 0)),
            pl.BlockSpec((R, D), lambda i: (i, 0)),
        ],
        out_shape=[
            jax.ShapeDtypeStruct((N, D), jnp.float32),
            jax.ShapeDtypeStruct((N, D), jnp.float32),
        ],
    )(x, W_in, b_in, counts_t, Wg1)


def _tc_mid(s, g, dinvb, b, W):
    def body(s_r, g_r, d_r, b_r, w_r, o_r):
        t = s_r[0] + s_r[1] + g_r[...]
        h = jnp.maximum(d_r[...] * t + b_r[...][None, :], 0.0)
        o_r[...] = (h @ w_r[...]) * d_r[...]

    return pl.pallas_call(
        body,
        grid=(G,),
        in_specs=[
            pl.BlockSpec((2, R, D), lambda i: (0, i, 0)),
            pl.BlockSpec((R, D), lambda i: (i, 0)),
            pl.BlockSpec((R, D), lambda i: (i, 0)),
            pl.BlockSpec((D,), lambda i: (0,)),
            pl.BlockSpec((D, D), lambda i: (0, 0)),
        ],
        out_specs=pl.BlockSpec((R, D), lambda i: (i, 0)),
        out_shape=jax.ShapeDtypeStruct((N, D), jnp.float32),
    )(s, g, dinvb, b, W)


def _tc_head(s, g, dinvb, bg3, src_ids, snk_ids, Wc1, bc1, Wc2, bc2, Wc3p,
             bc3p):
    def body(s_r, g_r, d_r, b3_r, sid_r, kid_r, w1_r, b1_r, w2_r, b2_r, w3_r,
             b3p_r, o_r, h4_ref, pair_ref):
        i = pl.program_id(0)

        @pl.when(i < G)
        def _():
            t = s_r[0] + s_r[1] + g_r[...]
            h4 = jnp.maximum(d_r[...] * t + b3_r[...][None, :], 0.0)
            h4_ref[pl.ds(i * R, R), :] = h4

        @pl.when(i == G)
        def _():
            for bb in range(B):
                si = sid_r[bb] + NPG * bb
                ki = kid_r[bb] + NPG * bb
                pair_ref[pl.ds(bb, 1), 0:D] = h4_ref[pl.ds(si, 1), :]
                pair_ref[pl.ds(bb, 1), D:2 * D] = h4_ref[pl.ds(ki, 1), :]
            pz = pair_ref[...]
            z1 = jnp.maximum(pz @ w1_r[...] + b1_r[...][None, :], 0.0)
            z2 = jnp.maximum(z1 @ w2_r[...] + b2_r[...][None, :], 0.0)
            o_r[...] = z2 @ w3_r[...] + b3p_r[...][None, :]

    cl = lambda i: (0, jnp.minimum(i, G - 1), 0)
    cl2 = lambda i: (jnp.minimum(i, G - 1), 0)
    return pl.pallas_call(
        body,
        grid=(G + 1,),
        in_specs=[
            pl.BlockSpec((2, R, D), cl),
            pl.BlockSpec((R, D), cl2),
            pl.BlockSpec((R, D), cl2),
            pl.BlockSpec((D,), lambda i: (0,)),
            pl.BlockSpec(memory_space=pltpu.SMEM),
            pl.BlockSpec(memory_space=pltpu.SMEM),
            pl.BlockSpec((2 * D, D), lambda i: (0, 0)),
            pl.BlockSpec((D,), lambda i: (0,)),
            pl.BlockSpec((D, D // 2), lambda i: (0, 0)),
            pl.BlockSpec((D // 2,), lambda i: (0,)),
            pl.BlockSpec((D // 2, D), lambda i: (0, 0)),
            pl.BlockSpec((D,), lambda i: (0,)),
        ],
        out_specs=pl.BlockSpec((B, D), lambda i: (0, 0)),
        out_shape=jax.ShapeDtypeStruct((B, D), jnp.float32),
        scratch_shapes=[
            pltpu.VMEM((N, D), jnp.float32),
            pltpu.VMEM((B, 2 * D), jnp.float32),
        ],
    )(s, g, dinvb, bg3, src_ids, snk_ids, Wc1, bc1, Wc2, bc2, Wc3p, bc3p)


def kernel(x, edge_index, batch, source_ids, sink_ids,
           W_in, b_in, Wg1, bg1, Wg2, bg2, Wg3, bg3,
           Wc1, bc1, Wc2, bc2, Wc3, bc3):
    srcw = jnp.pad(edge_index[0].reshape(NW, EPW),
                   ((0, 0), (0, EPWP - EPW)))
    dstw = jnp.pad(edge_index[1].reshape(NW, EPW),
                   ((0, 0), (0, EPWP - EPW)),
                   constant_values=NPAD - 1)
    cmb4 = jnp.concatenate([srcw.reshape(NW, NGRP, NBUF, K),
                            dstw.reshape(NW, NGRP, NBUF, K)], axis=2)
    onecol = jnp.zeros((K, D), jnp.float32).at[:, 0].set(1.0)
    z_rows = jnp.zeros((RB, D), jnp.float32)
    Wc3p = jnp.pad(Wc3, ((0, 0), (0, D - 2)))
    bc3p = jnp.pad(bc3, (0, D - 2))

    counts = _sc_degree(cmb4, onecol, z_rows)
    counts_t = counts[:, :N, 0].T

    g1, dinvb = _tc_pre(x, W_in, b_in, counts_t, Wg1)
    s1 = _sc_scatter(g1, cmb4, z_rows)
    g2 = _tc_mid(s1, g1, dinvb, bg1, Wg2)
    s2 = _sc_scatter(g2, cmb4, z_rows)
    g3 = _tc_mid(s2, g2, dinvb, bg2, Wg3)
    s3 = _sc_scatter(g3, cmb4, z_rows)
    out = _tc_head(s3, g3, dinvb, bg3, source_ids, sink_ids,
                   Wc1, bc1, Wc2, bc2, Wc3p, bc3p)
    return out[:, :2]


# speedup vs baseline: 1.8982x; 1.8982x over previous
import functools

import jax
import jax.numpy as jnp
from jax import lax
from jax.experimental import pallas as pl
from jax.experimental.pallas import tpu as pltpu
from jax.experimental.pallas import tpu_sc as plsc

N = 10000
E = 320000
D = 128
B = 8
NPG = N // B
NC = 2
NS = 16
NW = NC * NS
EPW = E // NW
EPWP = 10240
K = 64
NCHUNK = EPWP // K
NBUF = 4
NGRP = NCHUNK // NBUF
NB2 = NGRP // 2
KS = 40
NBS = 5
NGS = EPW // (KS * NBS)
NPAD = 10240
RPT = NPAD // NS
RB = 128
R = 1000
G = N // R


def _sc_mesh():
    return plsc.VectorSubcoreMesh(core_axis_name="c", subcore_axis_name="s")


def _sc_degree(cmb4, onecol, z_rows):
    @functools.partial(
        pl.kernel,
        out_type=jax.ShapeDtypeStruct((NC, NPAD, D), jnp.float32),
        mesh=_sc_mesh(),
        scratch_types=[
            pltpu.VMEM((K, D), jnp.float32),
            pltpu.VMEM((2 * NBUF, K), jnp.int32),
            pltpu.VMEM((2 * NBUF, K), jnp.int32),
            pltpu.VMEM_SHARED((NPAD, D), jnp.float32),
            pltpu.SemaphoreType.DMA,
        ]
        + [pltpu.SemaphoreType.DMA for _ in range(NBUF)],
    )
    def deg_kernel(cmb_h, oc_h, z_h, out_h, cbuf, ib0, ib1, acc, isem, *sems):
        c = lax.axis_index("c")
        s = lax.axis_index("s")
        wid = c * NS + s
        pltpu.sync_copy(oc_h, cbuf)
        base = s * RPT
        for j in range(RPT // RB):
            pltpu.sync_copy(z_h, acc.at[pl.ds(base + j * RB, RB)])
        plsc.subcore_barrier()

        def body(t, carry):
            pltpu.sync_copy(cmb_h.at[wid, 2 * t], ib0)
            dib1 = pltpu.async_copy(cmb_h.at[wid, 2 * t + 1], ib1, isem)
            d1 = [pltpu.async_copy(cbuf, acc.at[ib0.at[NBUF + b]], sems[b],
                                   add=True) for b in range(NBUF)]
            dib1.wait()
            for b in range(NBUF):
                d1[b].wait()
            d2 = [pltpu.async_copy(cbuf, acc.at[ib1.at[NBUF + b]], sems[b],
                                   add=True) for b in range(NBUF)]
            for b in range(NBUF):
                d2[b].wait()
            return carry

        lax.fori_loop(0, NB2, body, 0)
        plsc.subcore_barrier()
        pltpu.sync_copy(acc.at[pl.ds(base, RPT)], out_h.at[c, pl.ds(base, RPT)])

    return deg_kernel(cmb4, onecol, z_rows)


def _sc_scatter(g, cmbs, z_rows):
    @functools.partial(
        pl.kernel,
        out_type=jax.ShapeDtypeStruct((NC, NPAD, D), jnp.float32),
        mesh=_sc_mesh(),
        scratch_types=[
            pltpu.VMEM((2 * NBS, KS), jnp.int32),
            pltpu.VMEM_SHARED((NPAD, D), jnp.float32),
        ]
        + [pltpu.VMEM((KS, D), jnp.float32) for _ in range(NBS)]
        + [pltpu.SemaphoreType.DMA for _ in range(NBS)],
    )
    def scat_kernel(g_h, cmb_h, zr_h, out_h, ib, acc, *scr):
        rows = scr[:NBS]
        sems = scr[NBS:]
        c = lax.axis_index("c")
        s = lax.axis_index("s")
        wid = c * NS + s
        base = s * RPT
        for j in range(RPT // RB):
            pltpu.sync_copy(zr_h, acc.at[pl.ds(base + j * RB, RB)])
        plsc.subcore_barrier()

        def body(gi, carry):
            pltpu.sync_copy(cmb_h.at[wid, gi], ib)
            ga = [pltpu.async_copy(g_h.at[ib.at[b]], rows[b], sems[b])
                  for b in range(NBS)]
            for b in range(NBS):
                ga[b].wait()
            sa = [pltpu.async_copy(rows[b], acc.at[ib.at[NBS + b]],
                                   sems[b], add=True)
                  for b in range(NBS)]
            for b in range(NBS):
                sa[b].wait()
            return carry

        lax.fori_loop(0, NGS, body, 0)
        plsc.subcore_barrier()
        pltpu.sync_copy(acc.at[pl.ds(base, RPT)], out_h.at[c, pl.ds(base, RPT)])

    return scat_kernel(g, cmbs, z_rows)


def _tc_pre(x, W_in, b_in, counts_t, Wg1):
    def body(x_r, wi_r, bi_r, ct_r, wg_r, g_r, db_r):
        h = jnp.maximum(x_r[...] @ wi_r[...] + bi_r[...][None, :], 0.0)
        ct = ct_r[...]
        dcol = lax.rsqrt(ct[:, 0:1] + ct[:, 1:2] + 1.0)
        db_r[...] = jnp.broadcast_to(dcol, (R, D))
        g_r[...] = (h @ wg_r[...]) * dcol

    return pl.pallas_call(
        body,
        grid=(G,),
        in_specs=[
            pl.BlockSpec((R, D), lambda i: (i, 0)),
            pl.BlockSpec((D, D), lambda i: (0, 0)),
            pl.BlockSpec((D,), lambda i: (0,)),
            pl.BlockSpec((R, NC), lambda i: (i, 0)),
            pl.BlockSpec((D, D), lambda i: (0, 0)),
        ],
        out_specs=[
            pl.BlockSpec((R, D), lambda i: (i, 0)),
            pl.BlockSpec((R, D), lambda i: (i, 0)),
        ],
        out_shape=[
            jax.ShapeDtypeStruct((N, D), jnp.float32),
            jax.ShapeDtypeStruct((N, D), jnp.float32),
        ],
    )(x, W_in, b_in, counts_t, Wg1)


def _tc_mid(s, g, dinvb, b, W):
    def body(s_r, g_r, d_r, b_r, w_r, o_r):
        t = s_r[0] + s_r[1] + g_r[...]
        h = jnp.maximum(d_r[...] * t + b_r[...][None, :], 0.0)
        o_r[...] = (h @ w_r[...]) * d_r[...]

    return pl.pallas_call(
        body,
        grid=(G,),
        in_specs=[
            pl.BlockSpec((2, R, D), lambda i: (0, i, 0)),
            pl.BlockSpec((R, D), lambda i: (i, 0)),
            pl.BlockSpec((R, D), lambda i: (i, 0)),
            pl.BlockSpec((D,), lambda i: (0,)),
            pl.BlockSpec((D, D), lambda i: (0, 0)),
        ],
        out_specs=pl.BlockSpec((R, D), lambda i: (i, 0)),
        out_shape=jax.ShapeDtypeStruct((N, D), jnp.float32),
    )(s, g, dinvb, b, W)


def _tc_head(s, g, dinvb, bg3, src_ids, snk_ids, Wc1, bc1, Wc2, bc2, Wc3p,
             bc3p):
    def body(s_r, g_r, d_r, b3_r, sid_r, kid_r, w1_r, b1_r, w2_r, b2_r, w3_r,
             b3p_r, o_r, h4_ref, pair_ref):
        i = pl.program_id(0)

        @pl.when(i < G)
        def _():
            t = s_r[0] + s_r[1] + g_r[...]
            h4 = jnp.maximum(d_r[...] * t + b3_r[...][None, :], 0.0)
            h4_ref[pl.ds(i * R, R), :] = h4

        @pl.when(i == G)
        def _():
            for bb in range(B):
                si = sid_r[bb] + NPG * bb
                ki = kid_r[bb] + NPG * bb
                pair_ref[pl.ds(bb, 1), 0:D] = h4_ref[pl.ds(si, 1), :]
                pair_ref[pl.ds(bb, 1), D:2 * D] = h4_ref[pl.ds(ki, 1), :]
            pz = pair_ref[...]
            z1 = jnp.maximum(pz @ w1_r[...] + b1_r[...][None, :], 0.0)
            z2 = jnp.maximum(z1 @ w2_r[...] + b2_r[...][None, :], 0.0)
            o_r[...] = z2 @ w3_r[...] + b3p_r[...][None, :]

    cl = lambda i: (0, jnp.minimum(i, G - 1), 0)
    cl2 = lambda i: (jnp.minimum(i, G - 1), 0)
    return pl.pallas_call(
        body,
        grid=(G + 1,),
        in_specs=[
            pl.BlockSpec((2, R, D), cl),
            pl.BlockSpec((R, D), cl2),
            pl.BlockSpec((R, D), cl2),
            pl.BlockSpec((D,), lambda i: (0,)),
            pl.BlockSpec(memory_space=pltpu.SMEM),
            pl.BlockSpec(memory_space=pltpu.SMEM),
            pl.BlockSpec((2 * D, D), lambda i: (0, 0)),
            pl.BlockSpec((D,), lambda i: (0,)),
            pl.BlockSpec((D, D // 2), lambda i: (0, 0)),
            pl.BlockSpec((D // 2,), lambda i: (0,)),
            pl.BlockSpec((D // 2, D), lambda i: (0, 0)),
            pl.BlockSpec((D,), lambda i: (0,)),
        ],
        out_specs=pl.BlockSpec((B, D), lambda i: (0, 0)),
        out_shape=jax.ShapeDtypeStruct((B, D), jnp.float32),
        scratch_shapes=[
            pltpu.VMEM((N, D), jnp.float32),
            pltpu.VMEM((B, 2 * D), jnp.float32),
        ],
    )(s, g, dinvb, bg3, src_ids, snk_ids, Wc1, bc1, Wc2, bc2, Wc3p, bc3p)


def kernel(x, edge_index, batch, source_ids, sink_ids,
           W_in, b_in, Wg1, bg1, Wg2, bg2, Wg3, bg3,
           Wc1, bc1, Wc2, bc2, Wc3, bc3):
    srcw = jnp.pad(edge_index[0].reshape(NW, EPW),
                   ((0, 0), (0, EPWP - EPW)))
    dstw = jnp.pad(edge_index[1].reshape(NW, EPW),
                   ((0, 0), (0, EPWP - EPW)),
                   constant_values=NPAD - 1)
    cmb4 = jnp.concatenate([srcw.reshape(NW, NGRP, NBUF, K),
                            dstw.reshape(NW, NGRP, NBUF, K)], axis=2)
    cmbs = jnp.concatenate(
        [edge_index[0].reshape(NW, NGS, NBS, KS),
         edge_index[1].reshape(NW, NGS, NBS, KS)], axis=2)
    onecol = jnp.zeros((K, D), jnp.float32).at[:, 0].set(1.0)
    z_rows = jnp.zeros((RB, D), jnp.float32)
    Wc3p = jnp.pad(Wc3, ((0, 0), (0, D - 2)))
    bc3p = jnp.pad(bc3, (0, D - 2))

    counts = _sc_degree(cmb4, onecol, z_rows)
    counts_t = counts[:, :N, 0].T

    g1, dinvb = _tc_pre(x, W_in, b_in, counts_t, Wg1)
    s1 = _sc_scatter(g1, cmbs, z_rows)
    g2 = _tc_mid(s1, g1, dinvb, bg1, Wg2)
    s2 = _sc_scatter(g2, cmbs, z_rows)
    g3 = _tc_mid(s2, g2, dinvb, bg2, Wg3)
    s3 = _sc_scatter(g3, cmbs, z_rows)
    out = _tc_head(s3, g3, dinvb, bg3, source_ids, sink_ids,
                   Wc1, bc1, Wc2, bc2, Wc3p, bc3p)
    return out[:, :2]


# speedup vs baseline: 2.2002x; 1.1591x over previous
import functools

import jax
import jax.numpy as jnp
from jax import lax
from jax.experimental import pallas as pl
from jax.experimental.pallas import tpu as pltpu
from jax.experimental.pallas import tpu_sc as plsc

N = 10000
E = 320000
D = 128
B = 8
NPG = N // B
NC = 2
NS = 16
NW = NC * NS
EPW = E // NW
EPWP = 10240
K = 64
NCHUNK = EPWP // K
NBUF = 4
NGRP = NCHUNK // NBUF
NB2 = NGRP // 2
KS = 40
NBS = 5
NGS = EPW // (KS * NBS)
NPAD = 10240
RPT = NPAD // NS
RB = 128
R = 1000
G = N // R


def _sc_mesh():
    return plsc.VectorSubcoreMesh(core_axis_name="c", subcore_axis_name="s")


def _sc_degree(cmb4, onecol, z_rows):
    @functools.partial(
        pl.kernel,
        out_type=jax.ShapeDtypeStruct((NC, NPAD, D), jnp.float32),
        mesh=_sc_mesh(),
        scratch_types=[
            pltpu.VMEM((K, D), jnp.float32),
            pltpu.VMEM((2 * NBUF, K), jnp.int32),
            pltpu.VMEM((2 * NBUF, K), jnp.int32),
            pltpu.VMEM_SHARED((NPAD, D), jnp.float32),
            pltpu.SemaphoreType.DMA,
        ]
        + [pltpu.SemaphoreType.DMA for _ in range(NBUF)],
    )
    def deg_kernel(cmb_h, oc_h, z_h, out_h, cbuf, ib0, ib1, acc, isem, *sems):
        c = lax.axis_index("c")
        s = lax.axis_index("s")
        wid = c * NS + s
        pltpu.sync_copy(oc_h, cbuf)
        base = s * RPT
        for j in range(RPT // RB):
            pltpu.sync_copy(z_h, acc.at[pl.ds(base + j * RB, RB)])
        plsc.subcore_barrier()

        def body(t, carry):
            pltpu.sync_copy(cmb_h.at[wid, 2 * t], ib0)
            dib1 = pltpu.async_copy(cmb_h.at[wid, 2 * t + 1], ib1, isem)
            d1 = [pltpu.async_copy(cbuf, acc.at[ib0.at[NBUF + b]], sems[b],
                                   add=True) for b in range(NBUF)]
            dib1.wait()
            for b in range(NBUF):
                d1[b].wait()
            d2 = [pltpu.async_copy(cbuf, acc.at[ib1.at[NBUF + b]], sems[b],
                                   add=True) for b in range(NBUF)]
            for b in range(NBUF):
                d2[b].wait()
            return carry

        lax.fori_loop(0, NB2, body, 0)
        plsc.subcore_barrier()
        pltpu.sync_copy(acc.at[pl.ds(base, RPT)], out_h.at[c, pl.ds(base, RPT)])

    return deg_kernel(cmb4, onecol, z_rows)


def _sc_scatter(g, cmbs, z_rows):
    @functools.partial(
        pl.kernel,
        out_type=jax.ShapeDtypeStruct((NC, NPAD, D), jnp.float32),
        mesh=_sc_mesh(),
        scratch_types=[
            pltpu.VMEM((2 * NBS, KS), jnp.int32),
            pltpu.VMEM((2 * NBS, KS), jnp.int32),
            pltpu.VMEM_SHARED((NPAD, D), jnp.float32),
            pltpu.SemaphoreType.DMA,
        ]
        + [pltpu.VMEM((KS, D), jnp.float32) for _ in range(NBS)]
        + [pltpu.SemaphoreType.DMA for _ in range(NBS)],
    )
    def scat_kernel(g_h, cmb_h, zr_h, out_h, ib0, ib1, acc, isem, *scr):
        rows = scr[:NBS]
        sems = scr[NBS:]
        c = lax.axis_index("c")
        s = lax.axis_index("s")
        wid = c * NS + s
        base = s * RPT
        for j in range(RPT // RB):
            pltpu.sync_copy(zr_h, acc.at[pl.ds(base + j * RB, RB)])
        plsc.subcore_barrier()

        def body(t, carry):
            pltpu.sync_copy(cmb_h.at[wid, 2 * t], ib0)
            dib1 = pltpu.async_copy(cmb_h.at[wid, 2 * t + 1], ib1, isem)
            ga = [pltpu.async_copy(g_h.at[ib0.at[b]], rows[b], sems[b])
                  for b in range(NBS)]
            sa = []
            for b in range(NBS):
                ga[b].wait()
                sa.append(pltpu.async_copy(rows[b], acc.at[ib0.at[NBS + b]],
                                           sems[b], add=True))
            dib1.wait()
            gb = []
            for b in range(NBS):
                sa[b].wait()
                gb.append(pltpu.async_copy(g_h.at[ib1.at[b]], rows[b],
                                           sems[b]))
            sb = []
            for b in range(NBS):
                gb[b].wait()
                sb.append(pltpu.async_copy(rows[b], acc.at[ib1.at[NBS + b]],
                                           sems[b], add=True))
            for b in range(NBS):
                sb[b].wait()
            return carry

        lax.fori_loop(0, NGS // 2, body, 0)
        plsc.subcore_barrier()
        pltpu.sync_copy(acc.at[pl.ds(base, RPT)], out_h.at[c, pl.ds(base, RPT)])

    return scat_kernel(g, cmbs, z_rows)


def _tc_pre(x, W_in, b_in, counts_t, Wg1):
    def body(x_r, wi_r, bi_r, ct_r, wg_r, g_r, db_r):
        h = jnp.maximum(x_r[...] @ wi_r[...] + bi_r[...][None, :], 0.0)
        ct = ct_r[...]
        dcol = lax.rsqrt(ct[:, 0:1] + ct[:, 1:2] + 1.0)
        db_r[...] = jnp.broadcast_to(dcol, (R, D))
        g_r[...] = (h @ wg_r[...]) * dcol

    return pl.pallas_call(
        body,
        grid=(G,),
        in_specs=[
            pl.BlockSpec((R, D), lambda i: (i, 0)),
            pl.BlockSpec((D, D), lambda i: (0, 0)),
            pl.BlockSpec((D,), lambda i: (0,)),
            pl.BlockSpec((R, NC), lambda i: (i, 0)),
            pl.BlockSpec((D, D), lambda i: (0, 0)),
        ],
        out_specs=[
            pl.BlockSpec((R, D), lambda i: (i, 0)),
            pl.BlockSpec((R, D), lambda i: (i, 0)),
        ],
        out_shape=[
            jax.ShapeDtypeStruct((N, D), jnp.float32),
            jax.ShapeDtypeStruct((N, D), jnp.float32),
        ],
    )(x, W_in, b_in, counts_t, Wg1)


def _tc_mid(s, g, dinvb, b, W):
    def body(s_r, g_r, d_r, b_r, w_r, o_r):
        t = s_r[0] + s_r[1] + g_r[...]
        h = jnp.maximum(d_r[...] * t + b_r[...][None, :], 0.0)
        o_r[...] = (h @ w_r[...]) * d_r[...]

    return pl.pallas_call(
        body,
        grid=(G,),
        in_specs=[
            pl.BlockSpec((2, R, D), lambda i: (0, i, 0)),
            pl.BlockSpec((R, D), lambda i: (i, 0)),
            pl.BlockSpec((R, D), lambda i: (i, 0)),
            pl.BlockSpec((D,), lambda i: (0,)),
            pl.BlockSpec((D, D), lambda i: (0, 0)),
        ],
        out_specs=pl.BlockSpec((R, D), lambda i: (i, 0)),
        out_shape=jax.ShapeDtypeStruct((N, D), jnp.float32),
    )(s, g, dinvb, b, W)


def _tc_head(s, g, dinvb, bg3, src_ids, snk_ids, Wc1, bc1, Wc2, bc2, Wc3p,
             bc3p):
    def body(s_r, g_r, d_r, b3_r, sid_r, kid_r, w1_r, b1_r, w2_r, b2_r, w3_r,
             b3p_r, o_r, h4_ref, pair_ref):
        i = pl.program_id(0)

        @pl.when(i < G)
        def _():
            t = s_r[0] + s_r[1] + g_r[...]
            h4 = jnp.maximum(d_r[...] * t + b3_r[...][None, :], 0.0)
            h4_ref[pl.ds(i * R, R), :] = h4

        @pl.when(i == G)
        def _():
            for bb in range(B):
                si = sid_r[bb] + NPG * bb
                ki = kid_r[bb] + NPG * bb
                pair_ref[pl.ds(bb, 1), 0:D] = h4_ref[pl.ds(si, 1), :]
                pair_ref[pl.ds(bb, 1), D:2 * D] = h4_ref[pl.ds(ki, 1), :]
            pz = pair_ref[...]
            z1 = jnp.maximum(pz @ w1_r[...] + b1_r[...][None, :], 0.0)
            z2 = jnp.maximum(z1 @ w2_r[...] + b2_r[...][None, :], 0.0)
            o_r[...] = z2 @ w3_r[...] + b3p_r[...][None, :]

    cl = lambda i: (0, jnp.minimum(i, G - 1), 0)
    cl2 = lambda i: (jnp.minimum(i, G - 1), 0)
    return pl.pallas_call(
        body,
        grid=(G + 1,),
        in_specs=[
            pl.BlockSpec((2, R, D), cl),
            pl.BlockSpec((R, D), cl2),
            pl.BlockSpec((R, D), cl2),
            pl.BlockSpec((D,), lambda i: (0,)),
            pl.BlockSpec(memory_space=pltpu.SMEM),
            pl.BlockSpec(memory_space=pltpu.SMEM),
            pl.BlockSpec((2 * D, D), lambda i: (0, 0)),
            pl.BlockSpec((D,), lambda i: (0,)),
            pl.BlockSpec((D, D // 2), lambda i: (0, 0)),
            pl.BlockSpec((D // 2,), lambda i: (0,)),
            pl.BlockSpec((D // 2, D), lambda i: (0, 0)),
            pl.BlockSpec((D,), lambda i: (0,)),
        ],
        out_specs=pl.BlockSpec((B, D), lambda i: (0, 0)),
        out_shape=jax.ShapeDtypeStruct((B, D), jnp.float32),
        scratch_shapes=[
            pltpu.VMEM((N, D), jnp.float32),
            pltpu.VMEM((B, 2 * D), jnp.float32),
        ],
    )(s, g, dinvb, bg3, src_ids, snk_ids, Wc1, bc1, Wc2, bc2, Wc3p, bc3p)


def kernel(x, edge_index, batch, source_ids, sink_ids,
           W_in, b_in, Wg1, bg1, Wg2, bg2, Wg3, bg3,
           Wc1, bc1, Wc2, bc2, Wc3, bc3):
    srcw = jnp.pad(edge_index[0].reshape(NW, EPW),
                   ((0, 0), (0, EPWP - EPW)))
    dstw = jnp.pad(edge_index[1].reshape(NW, EPW),
                   ((0, 0), (0, EPWP - EPW)),
                   constant_values=NPAD - 1)
    cmb4 = jnp.concatenate([srcw.reshape(NW, NGRP, NBUF, K),
                            dstw.reshape(NW, NGRP, NBUF, K)], axis=2)
    cmbs = jnp.concatenate(
        [edge_index[0].reshape(NW, NGS, NBS, KS),
         edge_index[1].reshape(NW, NGS, NBS, KS)], axis=2)
    onecol = jnp.zeros((K, D), jnp.float32).at[:, 0].set(1.0)
    z_rows = jnp.zeros((RB, D), jnp.float32)
    Wc3p = jnp.pad(Wc3, ((0, 0), (0, D - 2)))
    bc3p = jnp.pad(bc3, (0, D - 2))

    counts = _sc_degree(cmb4, onecol, z_rows)
    counts_t = counts[:, :N, 0].T

    g1, dinvb = _tc_pre(x, W_in, b_in, counts_t, Wg1)
    s1 = _sc_scatter(g1, cmbs, z_rows)
    g2 = _tc_mid(s1, g1, dinvb, bg1, Wg2)
    s2 = _sc_scatter(g2, cmbs, z_rows)
    g3 = _tc_mid(s2, g2, dinvb, bg2, Wg3)
    s3 = _sc_scatter(g3, cmbs, z_rows)
    out = _tc_head(s3, g3, dinvb, bg3, source_ids, sink_ids,
                   Wc1, bc1, Wc2, bc2, Wc3p, bc3p)
    return out[:, :2]


# speedup vs baseline: 2.4785x; 1.1265x over previous
import functools

import jax
import jax.numpy as jnp
from jax import lax
from jax.experimental import pallas as pl
from jax.experimental.pallas import tpu as pltpu
from jax.experimental.pallas import tpu_sc as plsc

N = 10000
E = 320000
D = 128
B = 8
NPG = N // B
NC = 2
NS = 16
NW = NC * NS
EPW = E // NW
EPWP = 10240
K = 128
NCHUNK = EPWP // K
NBUF = 4
NGRP = NCHUNK // NBUF
NB2 = NGRP // 2
KS = 40
NBS = 5
NGS = EPW // (KS * NBS)
GPB = 5
NPAD = 10240
RPT = NPAD // NS
RB = 128
R = 1000
G = N // R


def _sc_mesh():
    return plsc.VectorSubcoreMesh(core_axis_name="c", subcore_axis_name="s")


def _sc_degree(cmb4, onecol, z_rows):
    @functools.partial(
        pl.kernel,
        out_type=jax.ShapeDtypeStruct((NC, NPAD, D), jnp.float32),
        mesh=_sc_mesh(),
        scratch_types=[
            pltpu.VMEM((K, D), jnp.float32),
            pltpu.VMEM((2 * NBUF, K), jnp.int32),
            pltpu.VMEM((2 * NBUF, K), jnp.int32),
            pltpu.VMEM_SHARED((NPAD, D), jnp.float32),
            pltpu.SemaphoreType.DMA,
        ]
        + [pltpu.SemaphoreType.DMA for _ in range(NBUF)],
    )
    def deg_kernel(cmb_h, oc_h, z_h, out_h, cbuf, ib0, ib1, acc, isem, *sems):
        c = lax.axis_index("c")
        s = lax.axis_index("s")
        wid = c * NS + s
        pltpu.sync_copy(oc_h, cbuf)
        base = s * RPT
        for j in range(RPT // RB):
            pltpu.sync_copy(z_h, acc.at[pl.ds(base + j * RB, RB)])
        plsc.subcore_barrier()

        def body(t, carry):
            pltpu.sync_copy(cmb_h.at[wid, 2 * t], ib0)
            dib1 = pltpu.async_copy(cmb_h.at[wid, 2 * t + 1], ib1, isem)
            d1 = [pltpu.async_copy(cbuf, acc.at[ib0.at[NBUF + b]], sems[b],
                                   add=True) for b in range(NBUF)]
            dib1.wait()
            for b in range(NBUF):
                d1[b].wait()
            d2 = [pltpu.async_copy(cbuf, acc.at[ib1.at[NBUF + b]], sems[b],
                                   add=True) for b in range(NBUF)]
            for b in range(NBUF):
                d2[b].wait()
            return carry

        lax.fori_loop(0, NB2, body, 0)
        plsc.subcore_barrier()
        pltpu.sync_copy(acc.at[pl.ds(base, RPT)], out_h.at[c, pl.ds(base, RPT)])

    return deg_kernel(cmb4, onecol, z_rows)


def _sc_scatter(g, cmbs, z_rows):
    @functools.partial(
        pl.kernel,
        out_type=jax.ShapeDtypeStruct((NC, NPAD, D), jnp.float32),
        mesh=_sc_mesh(),
        scratch_types=[pltpu.VMEM((2 * NBS, KS), jnp.int32)
                       for _ in range(GPB)]
        + [pltpu.VMEM_SHARED((NPAD, D), jnp.float32)]
        + [pltpu.SemaphoreType.DMA for _ in range(GPB - 1)]
        + [pltpu.VMEM((KS, D), jnp.float32) for _ in range(NBS)]
        + [pltpu.SemaphoreType.DMA for _ in range(NBS)],
    )
    def scat_kernel(g_h, cmb_h, zr_h, out_h, *scr):
        ibs = scr[:GPB]
        acc = scr[GPB]
        isems = scr[GPB + 1:GPB + GPB]
        rows = scr[GPB + GPB:GPB + GPB + NBS]
        sems = scr[GPB + GPB + NBS:]
        c = lax.axis_index("c")
        s = lax.axis_index("s")
        wid = c * NS + s
        base = s * RPT
        for j in range(RPT // RB):
            pltpu.sync_copy(zr_h, acc.at[pl.ds(base + j * RB, RB)])
        plsc.subcore_barrier()

        def body(t, carry):
            pltpu.sync_copy(cmb_h.at[wid, GPB * t], ibs[0])
            dpre = [pltpu.async_copy(cmb_h.at[wid, GPB * t + j], ibs[j],
                                     isems[j - 1])
                    for j in range(1, GPB)]
            prev = None
            for j in range(GPB):
                ib = ibs[j]
                if j > 0:
                    dpre[j - 1].wait()
                gd = []
                for b in range(NBS):
                    if prev is not None:
                        prev[b].wait()
                    gd.append(pltpu.async_copy(g_h.at[ib.at[b]], rows[b],
                                               sems[b]))
                sd = []
                for b in range(NBS):
                    gd[b].wait()
                    sd.append(pltpu.async_copy(rows[b],
                                               acc.at[ib.at[NBS + b]],
                                               sems[b], add=True))
                prev = sd
            for b in range(NBS):
                prev[b].wait()
            return carry

        lax.fori_loop(0, NGS // GPB, body, 0)
        plsc.subcore_barrier()
        pltpu.sync_copy(acc.at[pl.ds(base, RPT)], out_h.at[c, pl.ds(base, RPT)])

    return scat_kernel(g, cmbs, z_rows)


def _tc_pre(x, W_in, b_in, counts_t, Wg1):
    def body(x_r, wi_r, bi_r, ct_r, wg_r, g_r, db_r):
        h = jnp.maximum(x_r[...] @ wi_r[...] + bi_r[...][None, :], 0.0)
        ct = ct_r[...]
        dcol = lax.rsqrt(ct[:, 0:1] + ct[:, 1:2] + 1.0)
        db_r[...] = jnp.broadcast_to(dcol, (R, D))
        g_r[...] = (h @ wg_r[...]) * dcol

    return pl.pallas_call(
        body,
        grid=(G,),
        in_specs=[
            pl.BlockSpec((R, D), lambda i: (i, 0)),
            pl.BlockSpec((D, D), lambda i: (0, 0)),
            pl.BlockSpec((D,), lambda i: (0,)),
            pl.BlockSpec((R, NC), lambda i: (i, 0)),
            pl.BlockSpec((D, D), lambda i: (0, 0)),
        ],
        out_specs=[
            pl.BlockSpec((R, D), lambda i: (i, 0)),
            pl.BlockSpec((R, D), lambda i: (i, 0)),
        ],
        out_shape=[
            jax.ShapeDtypeStruct((N, D), jnp.float32),
            jax.ShapeDtypeStruct((N, D), jnp.float32),
        ],
    )(x, W_in, b_in, counts_t, Wg1)


def _tc_mid(s, g, dinvb, b, W):
    def body(s_r, g_r, d_r, b_r, w_r, o_r):
        t = s_r[0] + s_r[1] + g_r[...]
        h = jnp.maximum(d_r[...] * t + b_r[...][None, :], 0.0)
        o_r[...] = (h @ w_r[...]) * d_r[...]

    return pl.pallas_call(
        body,
        grid=(G,),
        in_specs=[
            pl.BlockSpec((2, R, D), lambda i: (0, i, 0)),
            pl.BlockSpec((R, D), lambda i: (i, 0)),
            pl.BlockSpec((R, D), lambda i: (i, 0)),
            pl.BlockSpec((D,), lambda i: (0,)),
            pl.BlockSpec((D, D), lambda i: (0, 0)),
        ],
        out_specs=pl.BlockSpec((R, D), lambda i: (i, 0)),
        out_shape=jax.ShapeDtypeStruct((N, D), jnp.float32),
    )(s, g, dinvb, b, W)


def _tc_head(s, g, dinvb, bg3, src_ids, snk_ids, Wc1, bc1, Wc2, bc2, Wc3p,
             bc3p):
    def body(s_r, g_r, d_r, b3_r, sid_r, kid_r, w1_r, b1_r, w2_r, b2_r, w3_r,
             b3p_r, o_r, h4_ref, pair_ref):
        i = pl.program_id(0)

        @pl.when(i < G)
        def _():
            t = s_r[0] + s_r[1] + g_r[...]
            h4 = jnp.maximum(d_r[...] * t + b3_r[...][None, :], 0.0)
            h4_ref[pl.ds(i * R, R), :] = h4

        @pl.when(i == G)
        def _():
            for bb in range(B):
                si = sid_r[bb] + NPG * bb
                ki = kid_r[bb] + NPG * bb
                pair_ref[pl.ds(bb, 1), 0:D] = h4_ref[pl.ds(si, 1), :]
                pair_ref[pl.ds(bb, 1), D:2 * D] = h4_ref[pl.ds(ki, 1), :]
            pz = pair_ref[...]
            z1 = jnp.maximum(pz @ w1_r[...] + b1_r[...][None, :], 0.0)
            z2 = jnp.maximum(z1 @ w2_r[...] + b2_r[...][None, :], 0.0)
            o_r[...] = z2 @ w3_r[...] + b3p_r[...][None, :]

    cl = lambda i: (0, jnp.minimum(i, G - 1), 0)
    cl2 = lambda i: (jnp.minimum(i, G - 1), 0)
    return pl.pallas_call(
        body,
        grid=(G + 1,),
        in_specs=[
            pl.BlockSpec((2, R, D), cl),
            pl.BlockSpec((R, D), cl2),
            pl.BlockSpec((R, D), cl2),
            pl.BlockSpec((D,), lambda i: (0,)),
            pl.BlockSpec(memory_space=pltpu.SMEM),
            pl.BlockSpec(memory_space=pltpu.SMEM),
            pl.BlockSpec((2 * D, D), lambda i: (0, 0)),
            pl.BlockSpec((D,), lambda i: (0,)),
            pl.BlockSpec((D, D // 2), lambda i: (0, 0)),
            pl.BlockSpec((D // 2,), lambda i: (0,)),
            pl.BlockSpec((D // 2, D), lambda i: (0, 0)),
            pl.BlockSpec((D,), lambda i: (0,)),
        ],
        out_specs=pl.BlockSpec((B, D), lambda i: (0, 0)),
        out_shape=jax.ShapeDtypeStruct((B, D), jnp.float32),
        scratch_shapes=[
            pltpu.VMEM((N, D), jnp.float32),
            pltpu.VMEM((B, 2 * D), jnp.float32),
        ],
    )(s, g, dinvb, bg3, src_ids, snk_ids, Wc1, bc1, Wc2, bc2, Wc3p, bc3p)


def kernel(x, edge_index, batch, source_ids, sink_ids,
           W_in, b_in, Wg1, bg1, Wg2, bg2, Wg3, bg3,
           Wc1, bc1, Wc2, bc2, Wc3, bc3):
    srcw = jnp.pad(edge_index[0].reshape(NW, EPW),
                   ((0, 0), (0, EPWP - EPW)))
    dstw = jnp.pad(edge_index[1].reshape(NW, EPW),
                   ((0, 0), (0, EPWP - EPW)),
                   constant_values=NPAD - 1)
    cmb4 = jnp.concatenate([srcw.reshape(NW, NGRP, NBUF, K),
                            dstw.reshape(NW, NGRP, NBUF, K)], axis=2)
    cmbs = jnp.concatenate(
        [edge_index[0].reshape(NW, NGS, NBS, KS),
         edge_index[1].reshape(NW, NGS, NBS, KS)], axis=2)
    onecol = jnp.zeros((K, D), jnp.float32).at[:, 0].set(1.0)
    z_rows = jnp.zeros((RB, D), jnp.float32)
    Wc3p = jnp.pad(Wc3, ((0, 0), (0, D - 2)))
    bc3p = jnp.pad(bc3, (0, D - 2))

    counts = _sc_degree(cmb4, onecol, z_rows)
    counts_t = counts[:, :N, 0].T

    g1, dinvb = _tc_pre(x, W_in, b_in, counts_t, Wg1)
    s1 = _sc_scatter(g1, cmbs, z_rows)
    g2 = _tc_mid(s1, g1, dinvb, bg1, Wg2)
    s2 = _sc_scatter(g2, cmbs, z_rows)
    g3 = _tc_mid(s2, g2, dinvb, bg2, Wg3)
    s3 = _sc_scatter(g3, cmbs, z_rows)
    out = _tc_head(s3, g3, dinvb, bg3, source_ids, sink_ids,
                   Wc1, bc1, Wc2, bc2, Wc3p, bc3p)
    return out[:, :2]


# speedup vs baseline: 2.4844x; 1.0024x over previous
import functools

import jax
import jax.numpy as jnp
from jax import lax
from jax.experimental import pallas as pl
from jax.experimental.pallas import tpu as pltpu
from jax.experimental.pallas import tpu_sc as plsc

N = 10000
E = 320000
D = 128
B = 8
NPG = N // B
NC = 2
NS = 16
NW = NC * NS
EPW = E // NW
EPWP = 10240
K = 128
NCHUNK = EPWP // K
NBUF = 4
NGRP = NCHUNK // NBUF
NB2 = NGRP // 2
KS = 40
NBS = 5
NGS = EPW // (KS * NBS)
GPB = 5
NPAD = 10240
RPT = NPAD // NS
RB = 128
R = 1000
G = N // R


def _sc_mesh():
    return plsc.VectorSubcoreMesh(core_axis_name="c", subcore_axis_name="s")


def _sc_degree(cmb4, onecol, z_rows):
    @functools.partial(
        pl.kernel,
        out_type=jax.ShapeDtypeStruct((NC, NPAD, D), jnp.float32),
        mesh=_sc_mesh(),
        scratch_types=[
            pltpu.VMEM((K, D), jnp.float32),
            pltpu.VMEM((2 * NBUF, K), jnp.int32),
            pltpu.VMEM((2 * NBUF, K), jnp.int32),
            pltpu.VMEM_SHARED((NPAD, D), jnp.float32),
            pltpu.SemaphoreType.DMA,
        ]
        + [pltpu.SemaphoreType.DMA for _ in range(NBUF)],
    )
    def deg_kernel(cmb_h, oc_h, z_h, out_h, cbuf, ib0, ib1, acc, isem, *sems):
        c = lax.axis_index("c")
        s = lax.axis_index("s")
        wid = c * NS + s
        pltpu.sync_copy(oc_h, cbuf)
        base = s * RPT
        for j in range(RPT // RB):
            pltpu.sync_copy(z_h, acc.at[pl.ds(base + j * RB, RB)])
        plsc.subcore_barrier()

        def body(t, carry):
            pltpu.sync_copy(cmb_h.at[wid, 2 * t], ib0)
            dib1 = pltpu.async_copy(cmb_h.at[wid, 2 * t + 1], ib1, isem)
            d1 = [pltpu.async_copy(cbuf, acc.at[ib0.at[NBUF + b]], sems[b],
                                   add=True) for b in range(NBUF)]
            dib1.wait()
            for b in range(NBUF):
                d1[b].wait()
            d2 = [pltpu.async_copy(cbuf, acc.at[ib1.at[NBUF + b]], sems[b],
                                   add=True) for b in range(NBUF)]
            for b in range(NBUF):
                d2[b].wait()
            return carry

        lax.fori_loop(0, NB2, body, 0)
        plsc.subcore_barrier()
        pltpu.sync_copy(acc.at[pl.ds(base, RPT)], out_h.at[c, pl.ds(base, RPT)])

    return deg_kernel(cmb4, onecol, z_rows)


def _sc_scatter(g, cmbs, z_rows):
    @functools.partial(
        pl.kernel,
        out_type=jax.ShapeDtypeStruct((NC, NPAD, D), jnp.float32),
        mesh=_sc_mesh(),
        scratch_types=[pltpu.VMEM((2 * NBS, KS), jnp.int32)
                       for _ in range(GPB)]
        + [pltpu.VMEM_SHARED((NPAD, D), jnp.float32)]
        + [pltpu.SemaphoreType.DMA for _ in range(GPB - 1)]
        + [pltpu.VMEM((KS, D), jnp.float32) for _ in range(NBS)]
        + [pltpu.SemaphoreType.DMA for _ in range(NBS)],
    )
    def scat_kernel(g_h, cmb_h, zr_h, out_h, *scr):
        ibs = scr[:GPB]
        acc = scr[GPB]
        isems = scr[GPB + 1:GPB + GPB]
        rows = scr[GPB + GPB:GPB + GPB + NBS]
        sems = scr[GPB + GPB + NBS:]
        c = lax.axis_index("c")
        s = lax.axis_index("s")
        wid = c * NS + s
        base = s * RPT
        for j in range(RPT // RB):
            pltpu.sync_copy(zr_h, acc.at[pl.ds(base + j * RB, RB)])
        plsc.subcore_barrier()

        def body(t, carry):
            pltpu.sync_copy(cmb_h.at[wid, GPB * t], ibs[0])
            dpre = [pltpu.async_copy(cmb_h.at[wid, GPB * t + j], ibs[j],
                                     isems[j - 1])
                    for j in range(1, GPB)]
            prev = None
            for j in range(GPB):
                ib = ibs[j]
                if j > 0:
                    dpre[j - 1].wait()
                gd = []
                for b in range(NBS):
                    if prev is not None:
                        prev[b].wait()
                    gd.append(pltpu.async_copy(g_h.at[ib.at[b]], rows[b],
                                               sems[b]))
                sd = []
                for b in range(NBS):
                    gd[b].wait()
                    sd.append(pltpu.async_copy(rows[b],
                                               acc.at[ib.at[NBS + b]],
                                               sems[b], add=True))
                prev = sd
            for b in range(NBS):
                prev[b].wait()
            return carry

        lax.fori_loop(0, NGS // GPB, body, 0)
        plsc.subcore_barrier()
        pltpu.sync_copy(acc.at[pl.ds(base, RPT)], out_h.at[c, pl.ds(base, RPT)])

    return scat_kernel(g, cmbs, z_rows)


def _tc_h1(x, W_in, b_in):
    def body(x_r, wi_r, bi_r, h_r):
        h_r[...] = jnp.maximum(x_r[...] @ wi_r[...] + bi_r[...][None, :], 0.0)

    return pl.pallas_call(
        body,
        grid=(G,),
        in_specs=[
            pl.BlockSpec((R, D), lambda i: (i, 0)),
            pl.BlockSpec((D, D), lambda i: (0, 0)),
            pl.BlockSpec((D,), lambda i: (0,)),
        ],
        out_specs=pl.BlockSpec((R, D), lambda i: (i, 0)),
        out_shape=jax.ShapeDtypeStruct((N, D), jnp.float32),
    )(x, W_in, b_in)


def _tc_g1(h1, counts_t, Wg1):
    def body(h_r, ct_r, wg_r, g_r, db_r):
        ct = ct_r[...]
        dcol = lax.rsqrt(ct[:, 0:1] + ct[:, 1:2] + 1.0)
        db_r[...] = jnp.broadcast_to(dcol, (R, D))
        g_r[...] = (h_r[...] @ wg_r[...]) * dcol

    return pl.pallas_call(
        body,
        grid=(G,),
        in_specs=[
            pl.BlockSpec((R, D), lambda i: (i, 0)),
            pl.BlockSpec((R, NC), lambda i: (i, 0)),
            pl.BlockSpec((D, D), lambda i: (0, 0)),
        ],
        out_specs=[
            pl.BlockSpec((R, D), lambda i: (i, 0)),
            pl.BlockSpec((R, D), lambda i: (i, 0)),
        ],
        out_shape=[
            jax.ShapeDtypeStruct((N, D), jnp.float32),
            jax.ShapeDtypeStruct((N, D), jnp.float32),
        ],
    )(h1, counts_t, Wg1)


def _tc_mid(s, g, dinvb, b, W):
    def body(s_r, g_r, d_r, b_r, w_r, o_r):
        t = s_r[0] + s_r[1] + g_r[...]
        h = jnp.maximum(d_r[...] * t + b_r[...][None, :], 0.0)
        o_r[...] = (h @ w_r[...]) * d_r[...]

    return pl.pallas_call(
        body,
        grid=(G,),
        in_specs=[
            pl.BlockSpec((2, R, D), lambda i: (0, i, 0)),
            pl.BlockSpec((R, D), lambda i: (i, 0)),
            pl.BlockSpec((R, D), lambda i: (i, 0)),
            pl.BlockSpec((D,), lambda i: (0,)),
            pl.BlockSpec((D, D), lambda i: (0, 0)),
        ],
        out_specs=pl.BlockSpec((R, D), lambda i: (i, 0)),
        out_shape=jax.ShapeDtypeStruct((N, D), jnp.float32),
    )(s, g, dinvb, b, W)


def _tc_head(s, g, dinvb, bg3, src_ids, snk_ids, Wc1, bc1, Wc2, bc2, Wc3p,
             bc3p):
    def body(s_r, g_r, d_r, b3_r, sid_r, kid_r, w1_r, b1_r, w2_r, b2_r, w3_r,
             b3p_r, o_r, h4_ref, pair_ref):
        i = pl.program_id(0)

        @pl.when(i < G)
        def _():
            t = s_r[0] + s_r[1] + g_r[...]
            h4 = jnp.maximum(d_r[...] * t + b3_r[...][None, :], 0.0)
            h4_ref[pl.ds(i * R, R), :] = h4

        @pl.when(i == G)
        def _():
            for bb in range(B):
                si = sid_r[bb] + NPG * bb
                ki = kid_r[bb] + NPG * bb
                pair_ref[pl.ds(bb, 1), 0:D] = h4_ref[pl.ds(si, 1), :]
                pair_ref[pl.ds(bb, 1), D:2 * D] = h4_ref[pl.ds(ki, 1), :]
            pz = pair_ref[...]
            z1 = jnp.maximum(pz @ w1_r[...] + b1_r[...][None, :], 0.0)
            z2 = jnp.maximum(z1 @ w2_r[...] + b2_r[...][None, :], 0.0)
            o_r[...] = z2 @ w3_r[...] + b3p_r[...][None, :]

    cl = lambda i: (0, jnp.minimum(i, G - 1), 0)
    cl2 = lambda i: (jnp.minimum(i, G - 1), 0)
    return pl.pallas_call(
        body,
        grid=(G + 1,),
        in_specs=[
            pl.BlockSpec((2, R, D), cl),
            pl.BlockSpec((R, D), cl2),
            pl.BlockSpec((R, D), cl2),
            pl.BlockSpec((D,), lambda i: (0,)),
            pl.BlockSpec(memory_space=pltpu.SMEM),
            pl.BlockSpec(memory_space=pltpu.SMEM),
            pl.BlockSpec((2 * D, D), lambda i: (0, 0)),
            pl.BlockSpec((D,), lambda i: (0,)),
            pl.BlockSpec((D, D // 2), lambda i: (0, 0)),
            pl.BlockSpec((D // 2,), lambda i: (0,)),
            pl.BlockSpec((D // 2, D), lambda i: (0, 0)),
            pl.BlockSpec((D,), lambda i: (0,)),
        ],
        out_specs=pl.BlockSpec((B, D), lambda i: (0, 0)),
        out_shape=jax.ShapeDtypeStruct((B, D), jnp.float32),
        scratch_shapes=[
            pltpu.VMEM((N, D), jnp.float32),
            pltpu.VMEM((B, 2 * D), jnp.float32),
        ],
    )(s, g, dinvb, bg3, src_ids, snk_ids, Wc1, bc1, Wc2, bc2, Wc3p, bc3p)


def kernel(x, edge_index, batch, source_ids, sink_ids,
           W_in, b_in, Wg1, bg1, Wg2, bg2, Wg3, bg3,
           Wc1, bc1, Wc2, bc2, Wc3, bc3):
    srcw = jnp.pad(edge_index[0].reshape(NW, EPW),
                   ((0, 0), (0, EPWP - EPW)))
    dstw = jnp.pad(edge_index[1].reshape(NW, EPW),
                   ((0, 0), (0, EPWP - EPW)),
                   constant_values=NPAD - 1)
    cmb4 = jnp.concatenate([srcw.reshape(NW, NGRP, NBUF, K),
                            dstw.reshape(NW, NGRP, NBUF, K)], axis=2)
    cmbs = jnp.concatenate(
        [edge_index[0].reshape(NW, NGS, NBS, KS),
         edge_index[1].reshape(NW, NGS, NBS, KS)], axis=2)
    onecol = jnp.zeros((K, D), jnp.float32).at[:, 0].set(1.0)
    z_rows = jnp.zeros((RB, D), jnp.float32)
    Wc3p = jnp.pad(Wc3, ((0, 0), (0, D - 2)))
    bc3p = jnp.pad(bc3, (0, D - 2))

    counts = _sc_degree(cmb4, onecol, z_rows)
    counts_t = counts[:, :N, 0].T

    h1 = _tc_h1(x, W_in, b_in)
    g1, dinvb = _tc_g1(h1, counts_t, Wg1)
    s1 = _sc_scatter(g1, cmbs, z_rows)
    g2 = _tc_mid(s1, g1, dinvb, bg1, Wg2)
    s2 = _sc_scatter(g2, cmbs, z_rows)
    g3 = _tc_mid(s2, g2, dinvb, bg2, Wg3)
    s3 = _sc_scatter(g3, cmbs, z_rows)
    out = _tc_head(s3, g3, dinvb, bg3, source_ids, sink_ids,
                   Wc1, bc1, Wc2, bc2, Wc3p, bc3p)
    return out[:, :2]


# speedup vs baseline: 2.5298x; 1.0183x over previous
import functools

import jax
import jax.numpy as jnp
from jax import lax
from jax.experimental import pallas as pl
from jax.experimental.pallas import tpu as pltpu
from jax.experimental.pallas import tpu_sc as plsc

N = 10000
E = 320000
D = 128
B = 8
NPG = N // B
NC = 2
NS = 16
NW = NC * NS
EPW = E // NW
EPWP = 10240
K = 128
NCHUNK = EPWP // K
NBUF = 4
NGRP = NCHUNK // NBUF
NB2 = NGRP // 2
KS = 40
NBS = 5
NGS = EPW // (KS * NBS)
GPB = 10
NPAD = 10240
RPT = NPAD // NS
RB = 128
R = 1000
G = N // R


def _sc_mesh():
    return plsc.VectorSubcoreMesh(core_axis_name="c", subcore_axis_name="s")


def _sc_degree(cmb4, onecol, z_rows):
    @functools.partial(
        pl.kernel,
        out_type=jax.ShapeDtypeStruct((NC, NPAD, D), jnp.float32),
        mesh=_sc_mesh(),
        scratch_types=[
            pltpu.VMEM((K, D), jnp.float32),
            pltpu.VMEM((2 * NBUF, K), jnp.int32),
            pltpu.VMEM((2 * NBUF, K), jnp.int32),
            pltpu.VMEM_SHARED((NPAD, D), jnp.float32),
            pltpu.SemaphoreType.DMA,
        ]
        + [pltpu.SemaphoreType.DMA for _ in range(NBUF)],
    )
    def deg_kernel(cmb_h, oc_h, z_h, out_h, cbuf, ib0, ib1, acc, isem, *sems):
        c = lax.axis_index("c")
        s = lax.axis_index("s")
        wid = c * NS + s
        pltpu.sync_copy(oc_h, cbuf)
        base = s * RPT
        for j in range(RPT // RB):
            pltpu.sync_copy(z_h, acc.at[pl.ds(base + j * RB, RB)])
        plsc.subcore_barrier()

        def body(t, carry):
            pltpu.sync_copy(cmb_h.at[wid, 2 * t], ib0)
            dib1 = pltpu.async_copy(cmb_h.at[wid, 2 * t + 1], ib1, isem)
            d1 = [pltpu.async_copy(cbuf, acc.at[ib0.at[NBUF + b]], sems[b],
                                   add=True) for b in range(NBUF)]
            dib1.wait()
            for b in range(NBUF):
                d1[b].wait()
            d2 = [pltpu.async_copy(cbuf, acc.at[ib1.at[NBUF + b]], sems[b],
                                   add=True) for b in range(NBUF)]
            for b in range(NBUF):
                d2[b].wait()
            return carry

        lax.fori_loop(0, NB2, body, 0)
        plsc.subcore_barrier()
        pltpu.sync_copy(acc.at[pl.ds(base, RPT)], out_h.at[c, pl.ds(base, RPT)])

    return deg_kernel(cmb4, onecol, z_rows)


def _sc_scatter(g, cmbs, z_rows):
    @functools.partial(
        pl.kernel,
        out_type=jax.ShapeDtypeStruct((NC, NPAD, D), jnp.float32),
        mesh=_sc_mesh(),
        scratch_types=[pltpu.VMEM((2 * NBS, KS), jnp.int32)
                       for _ in range(GPB)]
        + [pltpu.VMEM_SHARED((NPAD, D), jnp.float32)]
        + [pltpu.SemaphoreType.DMA for _ in range(GPB - 1)]
        + [pltpu.VMEM((KS, D), jnp.float32) for _ in range(NBS)]
        + [pltpu.SemaphoreType.DMA for _ in range(NBS)],
    )
    def scat_kernel(g_h, cmb_h, zr_h, out_h, *scr):
        ibs = scr[:GPB]
        acc = scr[GPB]
        isems = scr[GPB + 1:GPB + GPB]
        rows = scr[GPB + GPB:GPB + GPB + NBS]
        sems = scr[GPB + GPB + NBS:]
        c = lax.axis_index("c")
        s = lax.axis_index("s")
        wid = c * NS + s
        base = s * RPT
        for j in range(RPT // RB):
            pltpu.sync_copy(zr_h, acc.at[pl.ds(base + j * RB, RB)])
        plsc.subcore_barrier()

        def body(t, carry):
            pltpu.sync_copy(cmb_h.at[wid, GPB * t], ibs[0])
            dpre = [pltpu.async_copy(cmb_h.at[wid, GPB * t + j], ibs[j],
                                     isems[j - 1])
                    for j in range(1, GPB)]
            prev = None
            for j in range(GPB):
                ib = ibs[j]
                if j > 0:
                    dpre[j - 1].wait()
                gd = []
                for b in range(NBS):
                    if prev is not None:
                        prev[b].wait()
                    gd.append(pltpu.async_copy(g_h.at[ib.at[b]], rows[b],
                                               sems[b]))
                sd = []
                for b in range(NBS):
                    gd[b].wait()
                    sd.append(pltpu.async_copy(rows[b],
                                               acc.at[ib.at[NBS + b]],
                                               sems[b], add=True))
                prev = sd
            for b in range(NBS):
                prev[b].wait()
            return carry

        lax.fori_loop(0, NGS // GPB, body, 0)
        plsc.subcore_barrier()
        pltpu.sync_copy(acc.at[pl.ds(base, RPT)], out_h.at[c, pl.ds(base, RPT)])

    return scat_kernel(g, cmbs, z_rows)


def _tc_h1(x, W_in, b_in):
    def body(x_r, wi_r, bi_r, h_r):
        h_r[...] = jnp.maximum(x_r[...] @ wi_r[...] + bi_r[...][None, :], 0.0)

    return pl.pallas_call(
        body,
        grid=(G,),
        in_specs=[
            pl.BlockSpec((R, D), lambda i: (i, 0)),
            pl.BlockSpec((D, D), lambda i: (0, 0)),
            pl.BlockSpec((D,), lambda i: (0,)),
        ],
        out_specs=pl.BlockSpec((R, D), lambda i: (i, 0)),
        out_shape=jax.ShapeDtypeStruct((N, D), jnp.float32),
    )(x, W_in, b_in)


def _tc_g1(h1, counts_t, Wg1):
    def body(h_r, ct_r, wg_r, g_r, db_r):
        ct = ct_r[...]
        dcol = lax.rsqrt(ct[:, 0:1] + ct[:, 1:2] + 1.0)
        db_r[...] = jnp.broadcast_to(dcol, (R, D))
        g_r[...] = (h_r[...] @ wg_r[...]) * dcol

    return pl.pallas_call(
        body,
        grid=(G,),
        in_specs=[
            pl.BlockSpec((R, D), lambda i: (i, 0)),
            pl.BlockSpec((R, NC), lambda i: (i, 0)),
            pl.BlockSpec((D, D), lambda i: (0, 0)),
        ],
        out_specs=[
            pl.BlockSpec((R, D), lambda i: (i, 0)),
            pl.BlockSpec((R, D), lambda i: (i, 0)),
        ],
        out_shape=[
            jax.ShapeDtypeStruct((N, D), jnp.float32),
            jax.ShapeDtypeStruct((N, D), jnp.float32),
        ],
    )(h1, counts_t, Wg1)


def _tc_mid(s, g, dinvb, b, W):
    def body(s_r, g_r, d_r, b_r, w_r, o_r):
        t = s_r[0] + s_r[1] + g_r[...]
        h = jnp.maximum(d_r[...] * t + b_r[...][None, :], 0.0)
        o_r[...] = (h @ w_r[...]) * d_r[...]

    return pl.pallas_call(
        body,
        grid=(G,),
        in_specs=[
            pl.BlockSpec((2, R, D), lambda i: (0, i, 0)),
            pl.BlockSpec((R, D), lambda i: (i, 0)),
            pl.BlockSpec((R, D), lambda i: (i, 0)),
            pl.BlockSpec((D,), lambda i: (0,)),
            pl.BlockSpec((D, D), lambda i: (0, 0)),
        ],
        out_specs=pl.BlockSpec((R, D), lambda i: (i, 0)),
        out_shape=jax.ShapeDtypeStruct((N, D), jnp.float32),
    )(s, g, dinvb, b, W)


def _tc_head(s, g, dinvb, bg3, src_ids, snk_ids, Wc1, bc1, Wc2, bc2, Wc3p,
             bc3p):
    def body(s_r, g_r, d_r, b3_r, sid_r, kid_r, w1_r, b1_r, w2_r, b2_r, w3_r,
             b3p_r, o_r, h4_ref, pair_ref):
        i = pl.program_id(0)

        @pl.when(i < G)
        def _():
            t = s_r[0] + s_r[1] + g_r[...]
            h4 = jnp.maximum(d_r[...] * t + b3_r[...][None, :], 0.0)
            h4_ref[pl.ds(i * R, R), :] = h4

        @pl.when(i == G)
        def _():
            for bb in range(B):
                si = sid_r[bb] + NPG * bb
                ki = kid_r[bb] + NPG * bb
                pair_ref[pl.ds(bb, 1), 0:D] = h4_ref[pl.ds(si, 1), :]
                pair_ref[pl.ds(bb, 1), D:2 * D] = h4_ref[pl.ds(ki, 1), :]
            pz = pair_ref[...]
            z1 = jnp.maximum(pz @ w1_r[...] + b1_r[...][None, :], 0.0)
            z2 = jnp.maximum(z1 @ w2_r[...] + b2_r[...][None, :], 0.0)
            o_r[...] = z2 @ w3_r[...] + b3p_r[...][None, :]

    cl = lambda i: (0, jnp.minimum(i, G - 1), 0)
    cl2 = lambda i: (jnp.minimum(i, G - 1), 0)
    return pl.pallas_call(
        body,
        grid=(G + 1,),
        in_specs=[
            pl.BlockSpec((2, R, D), cl),
            pl.BlockSpec((R, D), cl2),
            pl.BlockSpec((R, D), cl2),
            pl.BlockSpec((D,), lambda i: (0,)),
            pl.BlockSpec(memory_space=pltpu.SMEM),
            pl.BlockSpec(memory_space=pltpu.SMEM),
            pl.BlockSpec((2 * D, D), lambda i: (0, 0)),
            pl.BlockSpec((D,), lambda i: (0,)),
            pl.BlockSpec((D, D // 2), lambda i: (0, 0)),
            pl.BlockSpec((D // 2,), lambda i: (0,)),
            pl.BlockSpec((D // 2, D), lambda i: (0, 0)),
            pl.BlockSpec((D,), lambda i: (0,)),
        ],
        out_specs=pl.BlockSpec((B, D), lambda i: (0, 0)),
        out_shape=jax.ShapeDtypeStruct((B, D), jnp.float32),
        scratch_shapes=[
            pltpu.VMEM((N, D), jnp.float32),
            pltpu.VMEM((B, 2 * D), jnp.float32),
        ],
    )(s, g, dinvb, bg3, src_ids, snk_ids, Wc1, bc1, Wc2, bc2, Wc3p, bc3p)


def kernel(x, edge_index, batch, source_ids, sink_ids,
           W_in, b_in, Wg1, bg1, Wg2, bg2, Wg3, bg3,
           Wc1, bc1, Wc2, bc2, Wc3, bc3):
    srcw = jnp.pad(edge_index[0].reshape(NW, EPW),
                   ((0, 0), (0, EPWP - EPW)))
    dstw = jnp.pad(edge_index[1].reshape(NW, EPW),
                   ((0, 0), (0, EPWP - EPW)),
                   constant_values=NPAD - 1)
    cmb4 = jnp.concatenate([srcw.reshape(NW, NGRP, NBUF, K),
                            dstw.reshape(NW, NGRP, NBUF, K)], axis=2)
    cmbs = jnp.concatenate(
        [edge_index[0].reshape(NW, NGS, NBS, KS),
         edge_index[1].reshape(NW, NGS, NBS, KS)], axis=2)
    onecol = jnp.zeros((K, D), jnp.float32).at[:, 0].set(1.0)
    z_rows = jnp.zeros((RB, D), jnp.float32)
    Wc3p = jnp.pad(Wc3, ((0, 0), (0, D - 2)))
    bc3p = jnp.pad(bc3, (0, D - 2))

    counts = _sc_degree(cmb4, onecol, z_rows)
    counts_t = counts[:, :N, 0].T

    h1 = _tc_h1(x, W_in, b_in)
    g1, dinvb = _tc_g1(h1, counts_t, Wg1)
    s1 = _sc_scatter(g1, cmbs, z_rows)
    g2 = _tc_mid(s1, g1, dinvb, bg1, Wg2)
    s2 = _sc_scatter(g2, cmbs, z_rows)
    g3 = _tc_mid(s2, g2, dinvb, bg2, Wg3)
    s3 = _sc_scatter(g3, cmbs, z_rows)
    out = _tc_head(s3, g3, dinvb, bg3, source_ids, sink_ids,
                   Wc1, bc1, Wc2, bc2, Wc3p, bc3p)
    return out[:, :2]
